# baseline (speedup 1.0000x reference)
import functools

import jax
import jax.numpy as jnp
from jax import lax
from jax.experimental import pallas as pl
from jax.experimental.pallas import tpu as pltpu
from jax.experimental.pallas import tpu_sc as plsc

N = 10000
E = 320000
D = 128

NC = 2
NS = 16
NW = NC * NS

CH = 128
NCH = 80
EPW = NCH * CH
E_PAD = NW * EPW
M0 = 56
M1 = 104
TCHUNK = NS * (M0 + M1)

NPAD = 10240
RPT = NPAD // NS
WB = RPT // CH

R = 400
GRID = N // R


def _sc_degree_body(dst3_hbm, ones_hbm, zeros_hbm, out_hbm, dst_all, ones_v, acc_sh, sem):
    cid = lax.axis_index("c")
    sid = lax.axis_index("s")
    wid = sid * NC + cid

    pltpu.sync_copy(ones_hbm, ones_v)
    pltpu.sync_copy(zeros_hbm.at[pl.ds(sid * RPT, RPT)],
                    acc_sh.at[pl.ds(sid * RPT, RPT)])

    pltpu.sync_copy(dst3_hbm.at[wid], dst_all)
    plsc.subcore_barrier()

    def body(c, carry):
        pltpu.sync_copy(ones_v, acc_sh.at[dst_all.at[c]], add=True)
        return carry

    lax.fori_loop(0, NCH, body, 0)
    plsc.subcore_barrier()

    pltpu.sync_copy(acc_sh.at[pl.ds(sid * RPT, RPT)],
                    out_hbm.at[cid, pl.ds(sid * RPT, RPT)])


def _sc_segsum_body(g_hbm, src3_hbm, dst3_hbm, zeros_hbm, out_hbm,
                    src_all, dst_all, rows_v, acc_sh, sem):
    cid = lax.axis_index("c")
    sid = lax.axis_index("s")
    wid = sid * NC + cid

    base = lax.select(cid == 0, sid * M0, NS * M0 + sid * M1)
    nmine = lax.select(cid == 0, M0, M1)

    @pl.when(cid == 0)
    def _():
        pltpu.sync_copy(src3_hbm.at[pl.ds(base, M0)], src_all.at[pl.ds(0, M0)])
        pltpu.sync_copy(dst3_hbm.at[pl.ds(base, M0)], dst_all.at[pl.ds(0, M0)])

    @pl.when(cid == 1)
    def _():
        pltpu.sync_copy(src3_hbm.at[pl.ds(base, M1)], src_all.at[pl.ds(0, M1)])
        pltpu.sync_copy(dst3_hbm.at[pl.ds(base, M1)], dst_all.at[pl.ds(0, M1)])

    pltpu.sync_copy(zeros_hbm.at[pl.ds(sid * RPT, RPT)],
                    acc_sh.at[pl.ds(sid * RPT, RPT)])
    plsc.subcore_barrier()

    def body(c, carry):
        pltpu.async_copy(g_hbm.at[src_all.at[c]], rows_v, sem).wait()
        pltpu.sync_copy(rows_v, acc_sh.at[dst_all.at[c]], add=True)
        return carry

    lax.fori_loop(0, nmine, body, 0)
    plsc.subcore_barrier()

    pltpu.sync_copy(acc_sh.at[pl.ds(sid * RPT, RPT)],
                    out_hbm.at[cid, pl.ds(sid * RPT, RPT)])


@functools.lru_cache(maxsize=None)
def _sc_kernels():
    mesh = plsc.VectorSubcoreMesh(core_axis_name="c", subcore_axis_name="s",
                                  num_cores=NC, num_subcores=NS)
    sc_degree = pl.kernel(
        _sc_degree_body,
        out_type=jax.ShapeDtypeStruct((NC, NPAD, D), jnp.float32),
        mesh=mesh,
        scratch_types=[
            pltpu.VMEM((NCH, CH), jnp.int32),
            pltpu.VMEM((CH, D), jnp.float32),
            pltpu.VMEM_SHARED((NPAD, D), jnp.float32),
            pltpu.SemaphoreType.DMA,
        ],
    )
    sc_segsum = pl.kernel(
        _sc_segsum_body,
        out_type=jax.ShapeDtypeStruct((NC, NPAD, D), jnp.float32),
        mesh=mesh,
        scratch_types=[
            pltpu.VMEM((M1, CH), jnp.int32),
            pltpu.VMEM((M1, CH), jnp.int32),
            pltpu.VMEM((CH, D), jnp.float32),
            pltpu.VMEM_SHARED((NPAD, D), jnp.float32),
            pltpu.SemaphoreType.DMA,
        ],
    )
    return sc_degree, sc_segsum


def _dinv(cA_ref, cB_ref):
    cnt = cA_ref[0, :, :1] + cB_ref[0, :, :1] + 1.0
    return lax.rsqrt(cnt)


def _t1_body(x_ref, w_ref, cA_ref, cB_ref, o_ref):
    o_ref[...] = jnp.dot(x_ref[...], w_ref[...],
                         preferred_element_type=jnp.float32) * _dinv(cA_ref, cB_ref)


def _t2_body(aA_ref, aB_ref, g_ref, cA_ref, cB_ref, b_ref, w_ref, o_ref):
    dinv = _dinv(cA_ref, cB_ref)
    h = dinv * (aA_ref[0] + aB_ref[0] + g_ref[...]) + b_ref[...]
    h = jnp.maximum(h, 0.0)
    o_ref[...] = jnp.dot(h, w_ref[...], preferred_element_type=jnp.float32) * dinv


def _t3_body(aA_ref, aB_ref, g_ref, cA_ref, cB_ref, b_ref, o_ref):
    dinv = _dinv(cA_ref, cB_ref)
    o_ref[...] = dinv * (aA_ref[0] + aB_ref[0] + g_ref[...]) + b_ref[...]


_row_spec = pl.BlockSpec((R, D), lambda i: (i, 0))
_w_spec = pl.BlockSpec((D, D), lambda i: (0, 0))
_b_spec = pl.BlockSpec((1, D), lambda i: (0, 0))
_accA_spec = pl.BlockSpec((1, R, D), lambda i: (0, i, 0))
_accB_spec = pl.BlockSpec((1, R, D), lambda i: (1, i, 0))
_cntA_spec = pl.BlockSpec((1, R, D), lambda i: (0, i, 0))
_cntB_spec = pl.BlockSpec((1, R, D), lambda i: (1, i, 0))
_out_shape = jax.ShapeDtypeStruct((N, D), jnp.float32)

_t1 = pl.pallas_call(
    _t1_body, grid=(GRID,), out_shape=_out_shape,
    in_specs=[_row_spec, _w_spec, _cntA_spec, _cntB_spec],
    out_specs=_row_spec,
)
_t2 = pl.pallas_call(
    _t2_body, grid=(GRID,), out_shape=_out_shape,
    in_specs=[_accA_spec, _accB_spec, _row_spec, _cntA_spec, _cntB_spec,
              _b_spec, _w_spec],
    out_specs=_row_spec,
)
_t3 = pl.pallas_call(
    _t3_body, grid=(GRID,), out_shape=_out_shape,
    in_specs=[_accA_spec, _accB_spec, _row_spec, _cntA_spec, _cntB_spec,
              _b_spec],
    out_specs=_row_spec,
)


@jax.jit
def kernel(features, edge_index, W1, b1, W2, b2):
    src = edge_index[0]
    dst = edge_index[1]
    pad = E_PAD - E
    srcp = jnp.concatenate([src, jnp.zeros((pad,), jnp.int32)])
    dstp = jnp.concatenate([dst, jnp.full((pad,), N, jnp.int32)])
    src2 = srcp.reshape(TCHUNK, CH)
    dst2 = dstp.reshape(TCHUNK, CH)
    dst3 = dstp.reshape(NW, NCH, CH)

    onesD = jnp.ones((CH, D), jnp.float32)
    zerosD = jnp.zeros((NPAD, D), jnp.float32)

    sc_degree, sc_segsum = _sc_kernels()
    cnt2 = sc_degree(dst3, onesD, zerosD)
    g1 = _t1(features, W1, cnt2, cnt2)
    acc1 = sc_segsum(g1, src2, dst2, zerosD)
    g2 = _t2(acc1, acc1, g1, cnt2, cnt2,
             b1.reshape(1, D), W2)
    acc2 = sc_segsum(g2, src2, dst2, zerosD)
    out = _t3(acc2, acc2, g2, cnt2, cnt2, b2.reshape(1, D))
    return out


# speedup vs baseline: 1.6084x; 1.6084x over previous
import functools

import jax
import jax.numpy as jnp
from jax import lax
from jax.experimental import pallas as pl
from jax.experimental.pallas import tpu as pltpu
from jax.experimental.pallas import tpu_sc as plsc

N = 10000
E = 320000
D = 128

NC = 2
NS = 16
NW = NC * NS

CH = 128
NCH = 79
EPW = NCH * CH
E_PAD = NW * EPW

NPAD = 10240
RPT = NPAD // NS
WB = RPT // CH

R = 400
GRID = N // R


def _sc_degree_body(dst3_hbm, ones_hbm, zeros_hbm, out_hbm, dst_all, ones_v, acc_sh, sem):
    cid = lax.axis_index("c")
    sid = lax.axis_index("s")
    wid = sid * NC + cid

    pltpu.sync_copy(ones_hbm, ones_v)
    pltpu.sync_copy(zeros_hbm.at[pl.ds(sid * RPT, RPT)],
                    acc_sh.at[pl.ds(sid * RPT, RPT)])

    pltpu.sync_copy(dst3_hbm.at[wid], dst_all)
    plsc.subcore_barrier()

    def body(c, carry):
        pltpu.sync_copy(ones_v, acc_sh.at[dst_all.at[c]], add=True)
        return carry

    lax.fori_loop(0, NCH, body, 0)
    plsc.subcore_barrier()

    pltpu.sync_copy(acc_sh.at[pl.ds(sid * RPT, RPT)],
                    out_hbm.at[cid, pl.ds(sid * RPT, RPT)])


def _sc_segsum_body(g_hbm, src3_hbm, dst3_hbm, zeros_hbm, out_hbm,
                    src_all, dst_all, rows_v, acc_sh, sem):
    cid = lax.axis_index("c")
    sid = lax.axis_index("s")
    wid = sid * NC + cid

    pltpu.sync_copy(src3_hbm.at[wid], src_all)
    pltpu.sync_copy(dst3_hbm.at[wid], dst_all)
    pltpu.sync_copy(zeros_hbm.at[pl.ds(sid * RPT, RPT)],
                    acc_sh.at[pl.ds(sid * RPT, RPT)])
    plsc.subcore_barrier()

    def body(c, carry):
        pltpu.async_copy(g_hbm.at[src_all.at[c]], rows_v, sem).wait()
        pltpu.sync_copy(rows_v, acc_sh.at[dst_all.at[c]], add=True)
        return carry

    lax.fori_loop(0, NCH, body, 0)
    plsc.subcore_barrier()

    pltpu.sync_copy(acc_sh.at[pl.ds(sid * RPT, RPT)],
                    out_hbm.at[cid, pl.ds(sid * RPT, RPT)])


@functools.lru_cache(maxsize=None)
def _sc_kernels():
    mesh = plsc.VectorSubcoreMesh(core_axis_name="c", subcore_axis_name="s",
                                  num_cores=NC, num_subcores=NS)
    sc_degree = pl.kernel(
        _sc_degree_body,
        out_type=jax.ShapeDtypeStruct((NC, NPAD, D), jnp.float32),
        mesh=mesh,
        scratch_types=[
            pltpu.VMEM((NCH, CH), jnp.int32),
            pltpu.VMEM((CH, D), jnp.float32),
            pltpu.VMEM_SHARED((NPAD, D), jnp.float32),
            pltpu.SemaphoreType.DMA,
        ],
    )
    sc_segsum = pl.kernel(
        _sc_segsum_body,
        out_type=jax.ShapeDtypeStruct((NC, NPAD, D), jnp.float32),
        mesh=mesh,
        scratch_types=[
            pltpu.VMEM((NCH, CH), jnp.int32),
            pltpu.VMEM((NCH, CH), jnp.int32),
            pltpu.VMEM((CH, D), jnp.float32),
            pltpu.VMEM_SHARED((NPAD, D), jnp.float32),
            pltpu.SemaphoreType.DMA,
        ],
    )
    return sc_degree, sc_segsum


def _dinv(cA_ref, cB_ref):
    cnt = cA_ref[0, :, :1] + cB_ref[0, :, :1] + 1.0
    return lax.rsqrt(cnt)


def _t1_body(x_ref, w_ref, cA_ref, cB_ref, o_ref):
    o_ref[...] = jnp.dot(x_ref[...], w_ref[...],
                         preferred_element_type=jnp.float32) * _dinv(cA_ref, cB_ref)


def _t2_body(aA_ref, aB_ref, g_ref, cA_ref, cB_ref, b_ref, w_ref, o_ref):
    dinv = _dinv(cA_ref, cB_ref)
    h = dinv * (aA_ref[0] + aB_ref[0] + g_ref[...]) + b_ref[...]
    h = jnp.maximum(h, 0.0)
    o_ref[...] = jnp.dot(h, w_ref[...], preferred_element_type=jnp.float32) * dinv


def _t3_body(aA_ref, aB_ref, g_ref, cA_ref, cB_ref, b_ref, o_ref):
    dinv = _dinv(cA_ref, cB_ref)
    o_ref[...] = dinv * (aA_ref[0] + aB_ref[0] + g_ref[...]) + b_ref[...]


_row_spec = pl.BlockSpec((R, D), lambda i: (i, 0))
_w_spec = pl.BlockSpec((D, D), lambda i: (0, 0))
_b_spec = pl.BlockSpec((1, D), lambda i: (0, 0))
_accA_spec = pl.BlockSpec((1, R, D), lambda i: (0, i, 0))
_accB_spec = pl.BlockSpec((1, R, D), lambda i: (1, i, 0))
_cntA_spec = pl.BlockSpec((1, R, D), lambda i: (0, i, 0))
_cntB_spec = pl.BlockSpec((1, R, D), lambda i: (1, i, 0))
_out_shape = jax.ShapeDtypeStruct((N, D), jnp.float32)

_t1 = pl.pallas_call(
    _t1_body, grid=(GRID,), out_shape=_out_shape,
    in_specs=[_row_spec, _w_spec, _cntA_spec, _cntB_spec],
    out_specs=_row_spec,
)
_t2 = pl.pallas_call(
    _t2_body, grid=(GRID,), out_shape=_out_shape,
    in_specs=[_accA_spec, _accB_spec, _row_spec, _cntA_spec, _cntB_spec,
              _b_spec, _w_spec],
    out_specs=_row_spec,
)
_t3 = pl.pallas_call(
    _t3_body, grid=(GRID,), out_shape=_out_shape,
    in_specs=[_accA_spec, _accB_spec, _row_spec, _cntA_spec, _cntB_spec,
              _b_spec],
    out_specs=_row_spec,
)


@jax.jit
def kernel(features, edge_index, W1, b1, W2, b2):
    src = edge_index[0]
    dst = edge_index[1]
    pad = E_PAD - E
    srcp = jnp.concatenate([src, jnp.zeros((pad,), jnp.int32)])
    dstp = jnp.concatenate([dst, jnp.full((pad,), N, jnp.int32)])
    src3 = srcp.reshape(NW, NCH, CH)
    dst3 = dstp.reshape(NW, NCH, CH)

    onesD = jnp.ones((CH, D), jnp.float32)
    zerosD = jnp.zeros((NPAD, D), jnp.float32)

    sc_degree, sc_segsum = _sc_kernels()
    cnt2 = sc_degree(dst3, onesD, zerosD)
    g1 = _t1(features, W1, cnt2, cnt2)
    acc1 = sc_segsum(g1, src3, dst3, zerosD)
    g2 = _t2(acc1, acc1, g1, cnt2, cnt2,
             b1.reshape(1, D), W2)
    acc2 = sc_segsum(g2, src3, dst3, zerosD)
    out = _t3(acc2, acc2, g2, cnt2, cnt2, b2.reshape(1, D))
    return out
